# Initial kernel scaffold; baseline (speedup 1.0000x reference)
import jax
import jax.numpy as jnp
from jax.experimental import pallas as pl


def kernel(speaker_id, speaker_table, proj_w, proj_b):
    raise NotImplementedError("write your pallas kernel here")


# speedup vs baseline: 1.0647x; 1.0647x over previous
import functools

import jax
import jax.numpy as jnp
from jax import lax
from jax.experimental import pallas as pl
from jax.experimental.pallas import tpu as pltpu
from jax.experimental.pallas import tpu_sc as plsc

N_SPEAKERS = 10000
EMBED = 512
HIDDEN = 1024
BATCH = 16384


_BM = 400


def _mm_body(a_ref, w_ref, b_ref, o_ref):
    o_ref[...] = (
        lax.dot_general(
            a_ref[...], w_ref[...],
            (((1,), (1,)), ((), ())),
            preferred_element_type=jnp.float32,
        )
        + b_ref[...]
    )


def _project_table(table, w, b2d):
    return pl.pallas_call(
        _mm_body,
        grid=(N_SPEAKERS // _BM,),
        in_specs=[
            pl.BlockSpec((_BM, EMBED), lambda i: (i, 0)),
            pl.BlockSpec((HIDDEN, EMBED), lambda i: (0, 0)),
            pl.BlockSpec((1, HIDDEN), lambda i: (0, 0)),
        ],
        out_specs=pl.BlockSpec((_BM, HIDDEN), lambda i: (i, 0)),
        out_shape=jax.ShapeDtypeStruct((N_SPEAKERS, HIDDEN), jnp.float32),
    )(table, w, b2d)


_NC = 2
_NS = 16
_NW = _NC * _NS
_B_PER_W = BATCH // _NW
_C = 32
_NCH = _B_PER_W // _C

_sc_mesh = plsc.VectorSubcoreMesh(core_axis_name="c", subcore_axis_name="s")


@functools.partial(
    pl.kernel,
    mesh=_sc_mesh,
    out_type=jax.ShapeDtypeStruct((BATCH, HIDDEN), jnp.float32),
    scratch_types=[
        pltpu.VMEM((_B_PER_W,), jnp.int32),
        pltpu.VMEM((_C, HIDDEN), jnp.float32),
        pltpu.VMEM((_C, HIDDEN), jnp.float32),
        pltpu.SemaphoreType.DMA,
        pltpu.SemaphoreType.DMA,
    ],
)
def _sc_gather(ids_hbm, ptab_hbm, out_hbm, idx_v, buf0, buf1, sem0, sem1):
    wid = lax.axis_index("s") * _NC + lax.axis_index("c")
    base = wid * _B_PER_W
    pltpu.sync_copy(ids_hbm.at[pl.ds(base, _B_PER_W)], idx_v)

    bufs = (buf0, buf1)
    sems = (sem0, sem1)

    def start_gather(c):
        return pltpu.async_copy(
            ptab_hbm.at[idx_v.at[pl.ds(c * _C, _C)]], bufs[c % 2], sems[c % 2]
        )

    pending = start_gather(0)
    for c in range(_NCH):
        nxt = start_gather(c + 1) if c + 1 < _NCH else None
        pending.wait()
        pltpu.sync_copy(bufs[c % 2], out_hbm.at[pl.ds(base + c * _C, _C)])
        pending = nxt


def kernel(speaker_id, speaker_table, proj_w, proj_b):
    ids = speaker_id.astype(jnp.int32)
    proj_table = _project_table(speaker_table, proj_w, proj_b.reshape(1, HIDDEN))
    return _sc_gather(ids, proj_table)


# speedup vs baseline: 1.1827x; 1.1108x over previous
import functools

import jax
import jax.numpy as jnp
from jax import lax
from jax.experimental import pallas as pl
from jax.experimental.pallas import tpu as pltpu
from jax.experimental.pallas import tpu_sc as plsc

N_SPEAKERS = 10000
EMBED = 512
HIDDEN = 1024
BATCH = 16384


_BM = 2000


def _mm_body(a_ref, w_ref, b_ref, o_ref):
    o_ref[...] = (
        lax.dot_general(
            a_ref[...], w_ref[...],
            (((1,), (1,)), ((), ())),
            preferred_element_type=jnp.float32,
        )
        + b_ref[...]
    )


def _project_table(table, w, b2d):
    return pl.pallas_call(
        _mm_body,
        grid=(N_SPEAKERS // _BM,),
        in_specs=[
            pl.BlockSpec((_BM, EMBED), lambda i: (i, 0)),
            pl.BlockSpec((HIDDEN, EMBED), lambda i: (0, 0)),
            pl.BlockSpec((1, HIDDEN), lambda i: (0, 0)),
        ],
        out_specs=pl.BlockSpec((_BM, HIDDEN), lambda i: (i, 0)),
        out_shape=jax.ShapeDtypeStruct((N_SPEAKERS, HIDDEN), jnp.float32),
    )(table, w, b2d)


_NC = 2
_NS = 16
_NW = _NC * _NS
_B_PER_W = BATCH // _NW
_C = 32
_NCH = _B_PER_W // _C
_NBUF = 3

_sc_mesh = plsc.VectorSubcoreMesh(core_axis_name="c", subcore_axis_name="s")


@functools.partial(
    pl.kernel,
    mesh=_sc_mesh,
    out_type=jax.ShapeDtypeStruct((BATCH, HIDDEN), jnp.float32),
    scratch_types=[
        pltpu.VMEM((_B_PER_W,), jnp.int32),
        pltpu.VMEM((_C, HIDDEN), jnp.float32),
        pltpu.VMEM((_C, HIDDEN), jnp.float32),
        pltpu.VMEM((_C, HIDDEN), jnp.float32),
        pltpu.SemaphoreType.DMA,
        pltpu.SemaphoreType.DMA,
        pltpu.SemaphoreType.DMA,
        pltpu.SemaphoreType.DMA,
        pltpu.SemaphoreType.DMA,
        pltpu.SemaphoreType.DMA,
    ],
)
def _sc_gather(ids_hbm, ptab_hbm, out_hbm, idx_v,
               buf0, buf1, buf2, sg0, sg1, sg2, ss0, ss1, ss2):
    wid = lax.axis_index("s") * _NC + lax.axis_index("c")
    base = wid * _B_PER_W
    pltpu.sync_copy(ids_hbm.at[pl.ds(base, _B_PER_W)], idx_v)

    bufs = (buf0, buf1, buf2)
    sg = (sg0, sg1, sg2)
    ss = (ss0, ss1, ss2)

    def start_gather(c):
        return pltpu.async_copy(
            ptab_hbm.at[idx_v.at[pl.ds(c * _C, _C)]], bufs[c % _NBUF], sg[c % _NBUF]
        )

    def start_scatter(c):
        return pltpu.async_copy(
            bufs[c % _NBUF], out_hbm.at[pl.ds(base + c * _C, _C)], ss[c % _NBUF]
        )

    gathers = [None] * _NCH
    scatters = [None] * _NCH
    gathers[0] = start_gather(0)
    gathers[1] = start_gather(1)
    for c in range(_NCH):
        gathers[c].wait()
        scatters[c] = start_scatter(c)
        nxt = c + 2
        if nxt < _NCH:
            if c >= 1:
                scatters[c - 1].wait()
            gathers[nxt] = start_gather(nxt)
    scatters[_NCH - 3].wait()
    scatters[_NCH - 2].wait()
    scatters[_NCH - 1].wait()


def kernel(speaker_id, speaker_table, proj_w, proj_b):
    ids = speaker_id.astype(jnp.int32)
    proj_table = _project_table(speaker_table, proj_w, proj_b.reshape(1, HIDDEN))
    return _sc_gather(ids, proj_table)
